# Initial kernel scaffold; baseline (speedup 1.0000x reference)
import jax
import jax.numpy as jnp
from jax.experimental import pallas as pl


def kernel(x, edge_index, Wk, bk, Wq, bq, Wv, bv, Ws, bs):
    raise NotImplementedError("write your pallas kernel here")


# speedup vs baseline: 5.4122x; 5.4122x over previous
import functools

import jax
import jax.numpy as jnp
from jax import lax
from jax.experimental import pallas as pl
from jax.experimental.pallas import tpu as pltpu
from jax.experimental.pallas import tpu_sc as plsc


def _mm3_body(x_ref, wk_ref, bk_ref, wq_ref, bq_ref, wv_ref, bv_ref,
              k_ref, q_ref, v_ref):
    xb = x_ref[...]
    k_ref[...] = jnp.dot(xb, wk_ref[...], preferred_element_type=jnp.float32) + bk_ref[...]
    q_ref[...] = jnp.dot(xb, wq_ref[...], preferred_element_type=jnp.float32) + bq_ref[...]
    v_ref[...] = jnp.dot(xb, wv_ref[...], preferred_element_type=jnp.float32) + bv_ref[...]


def _mm3(x, Wk, bk, Wq, bq, Wv, bv, rb):
    n, d = x.shape
    grid = (n // rb,)
    row_bs = pl.BlockSpec((rb, d), lambda i: (i, 0))
    w_bs = pl.BlockSpec((d, d), lambda i: (0, 0))
    b_bs = pl.BlockSpec((1, d), lambda i: (0, 0))
    out_sd = jax.ShapeDtypeStruct((n, d), jnp.float32)
    return pl.pallas_call(
        _mm3_body,
        grid=grid,
        in_specs=[row_bs, w_bs, b_bs, w_bs, b_bs, w_bs, b_bs],
        out_specs=[row_bs, row_bs, row_bs],
        out_shape=[out_sd, out_sd, out_sd],
    )(x, Wk, bk.reshape(1, d), Wq, bq.reshape(1, d), Wv, bv.reshape(1, d))


def _combine_body(x_ref, ws_ref, bs_ref, p_ref, o_ref):
    o_ref[...] = (jnp.dot(x_ref[...], ws_ref[...], preferred_element_type=jnp.float32)
                  + bs_ref[...] + p_ref[0] + p_ref[1])


def _combine(x, Ws, bs, partials, rb):
    n, d = x.shape
    grid = (n // rb,)
    return pl.pallas_call(
        _combine_body,
        grid=grid,
        in_specs=[
            pl.BlockSpec((rb, d), lambda i: (i, 0)),
            pl.BlockSpec((d, d), lambda i: (0, 0)),
            pl.BlockSpec((1, d), lambda i: (0, 0)),
            pl.BlockSpec((2, rb, d), lambda i: (0, i, 0)),
        ],
        out_specs=pl.BlockSpec((rb, d), lambda i: (i, 0)),
        out_shape=jax.ShapeDtypeStruct((n, d), jnp.float32),
    )(x, Ws, bs.reshape(1, d), partials)


def _edge_sc(k, q, v, src, dst):
    n, d = k.shape
    e = src.shape[0]
    nc, ns = 2, 16
    nw = nc * ns
    ept = e // nw
    C = 80
    nchunk = ept // C
    ZR = 104
    nz = 6
    rpt = ZR * nz
    tail = n - rpt * ns
    assert ept * nw == e and nchunk * C == ept
    assert 0 <= tail <= ZR and tail % 8 == 0 and d % 16 == 0

    mesh = plsc.VectorSubcoreMesh(core_axis_name="c", subcore_axis_name="s")

    @functools.partial(
        pl.kernel, mesh=mesh,
        out_type=jax.ShapeDtypeStruct((nc, n, d), jnp.float32),
        scratch_types=[
            pltpu.VMEM((C,), jnp.int32),
            pltpu.VMEM((C,), jnp.int32),
            pltpu.VMEM((C, d), jnp.float32),
            pltpu.VMEM((C, d), jnp.float32),
            pltpu.VMEM((C, d), jnp.float32),
            pltpu.VMEM((ZR, d), jnp.float32),
            pltpu.VMEM_SHARED((n, d), jnp.float32),
            pltpu.SemaphoreType.DMA,
            pltpu.SemaphoreType.DMA,
            pltpu.SemaphoreType.DMA,
        ],
    )
    def ek(k_hbm, q_hbm, v_hbm, src_hbm, dst_hbm, out_hbm,
           dst_v, src_v, kb, qb, vb, zb, agg, sem_k, sem_q, sem_v):
        cid = lax.axis_index("c")
        sid = lax.axis_index("s")
        w = cid * ns + sid

        zero16 = jnp.zeros((16,), jnp.float32)

        def zero_row(r, carry):
            for g in range(d // 16):
                zb[r, pl.ds(g * 16, 16)] = zero16
            return carry
        lax.fori_loop(0, ZR, zero_row, 0)

        for j in range(nz):
            pltpu.sync_copy(zb, agg.at[pl.ds(sid * rpt + j * ZR, ZR)])
        if tail:
            @pl.when(sid == ns - 1)
            def _():
                pltpu.sync_copy(zb.at[pl.ds(0, tail)],
                                agg.at[pl.ds(rpt * ns, tail)])
        plsc.subcore_barrier()

        def chunk(c, carry):
            base = w * ept + c * C
            pltpu.sync_copy(dst_hbm.at[pl.ds(base, C)], dst_v)
            pltpu.sync_copy(src_hbm.at[pl.ds(base, C)], src_v)
            cpk = pltpu.async_copy(k_hbm.at[dst_v], kb, sem_k)
            cpq = pltpu.async_copy(q_hbm.at[src_v], qb, sem_q)
            cpv = pltpu.async_copy(v_hbm.at[src_v], vb, sem_v)
            cpk.wait()
            cpq.wait()
            cpv.wait()

            def row(r, rcarry):
                for g in range(d // 16):
                    sl = pl.ds(g * 16, 16)
                    z = kb[r, sl] + qb[r, sl]
                    vb[r, sl] = vb[r, sl] / (1.0 + jnp.exp(-z))
                return rcarry
            lax.fori_loop(0, C, row, 0)

            pltpu.sync_copy(vb, agg.at[dst_v], add=True)
            return carry
        lax.fori_loop(0, nchunk, chunk, 0)

        plsc.subcore_barrier()

        for j in range(nz):
            off = sid * rpt + j * ZR
            pltpu.sync_copy(agg.at[pl.ds(off, ZR)], zb)
            pltpu.sync_copy(zb, out_hbm.at[cid, pl.ds(off, ZR)])
        if tail:
            @pl.when(sid == ns - 1)
            def _():
                pltpu.sync_copy(agg.at[pl.ds(rpt * ns, tail)],
                                zb.at[pl.ds(0, tail)])
                pltpu.sync_copy(zb.at[pl.ds(0, tail)],
                                out_hbm.at[cid, pl.ds(rpt * ns, tail)])

    return ek(k, q, v, src, dst)


def kernel(x, edge_index, Wk, bk, Wq, bq, Wv, bv, Ws, bs):
    n, d = x.shape
    k, q, v = _mm3(x, Wk, bk, Wq, bq, Wv, bv, 400)
    partials = _edge_sc(k, q, v, edge_index[0], edge_index[1])
    return _combine(x, Ws, bs, partials, 400)


# speedup vs baseline: 6.5992x; 1.2193x over previous
import functools

import jax
import jax.numpy as jnp
from jax import lax
from jax.experimental import pallas as pl
from jax.experimental.pallas import tpu as pltpu
from jax.experimental.pallas import tpu_sc as plsc


def _mm3_body(x_ref, wk_ref, bk_ref, wq_ref, bq_ref, wv_ref, bv_ref,
              k_ref, q_ref, v_ref):
    xb = x_ref[...]
    k_ref[...] = jnp.dot(xb, wk_ref[...], preferred_element_type=jnp.float32) + bk_ref[...]
    q_ref[...] = jnp.dot(xb, wq_ref[...], preferred_element_type=jnp.float32) + bq_ref[...]
    v_ref[...] = jnp.dot(xb, wv_ref[...], preferred_element_type=jnp.float32) + bv_ref[...]


def _mm3(x, Wk, bk, Wq, bq, Wv, bv, rb):
    n, d = x.shape
    grid = (n // rb,)
    row_bs = pl.BlockSpec((rb, d), lambda i: (i, 0))
    w_bs = pl.BlockSpec((d, d), lambda i: (0, 0))
    b_bs = pl.BlockSpec((1, d), lambda i: (0, 0))
    out_sd = jax.ShapeDtypeStruct((n, d), jnp.float32)
    return pl.pallas_call(
        _mm3_body,
        grid=grid,
        in_specs=[row_bs, w_bs, b_bs, w_bs, b_bs, w_bs, b_bs],
        out_specs=[row_bs, row_bs, row_bs],
        out_shape=[out_sd, out_sd, out_sd],
    )(x, Wk, bk.reshape(1, d), Wq, bq.reshape(1, d), Wv, bv.reshape(1, d))


def _combine_body(x_ref, ws_ref, bs_ref, p_ref, o_ref):
    o_ref[...] = (jnp.dot(x_ref[...], ws_ref[...], preferred_element_type=jnp.float32)
                  + bs_ref[...] + p_ref[0] + p_ref[1])


def _combine(x, Ws, bs, partials, rb):
    n, d = x.shape
    grid = (n // rb,)
    return pl.pallas_call(
        _combine_body,
        grid=grid,
        in_specs=[
            pl.BlockSpec((rb, d), lambda i: (i, 0)),
            pl.BlockSpec((d, d), lambda i: (0, 0)),
            pl.BlockSpec((1, d), lambda i: (0, 0)),
            pl.BlockSpec((2, rb, d), lambda i: (0, i, 0)),
        ],
        out_specs=pl.BlockSpec((rb, d), lambda i: (i, 0)),
        out_shape=jax.ShapeDtypeStruct((n, d), jnp.float32),
    )(x, Ws, bs.reshape(1, d), partials)


def _edge_sc(k, q, v, src, dst):
    n, d = k.shape
    e = src.shape[0]
    nc, ns = 2, 16
    nw = nc * ns
    ept = e // nw
    C = 40
    nchunk = ept // C
    npair = nchunk // 2
    ZR = 104
    nz = 6
    rpt = ZR * nz
    tail = n - rpt * ns
    assert ept * nw == e and nchunk * C == ept and npair * 2 == nchunk
    assert 0 <= tail <= ZR and tail % 8 == 0 and d % 16 == 0 and C % 8 == 0

    mesh = plsc.VectorSubcoreMesh(core_axis_name="c", subcore_axis_name="s")

    @functools.partial(
        pl.kernel, mesh=mesh,
        out_type=jax.ShapeDtypeStruct((nc, n, d), jnp.float32),
        scratch_types=[
            pltpu.VMEM((2, C), jnp.int32),
            pltpu.VMEM((2, C), jnp.int32),
            pltpu.VMEM((2, C, d), jnp.float32),
            pltpu.VMEM((2, C, d), jnp.float32),
            pltpu.VMEM((2, C, d), jnp.float32),
            pltpu.VMEM((ZR, d), jnp.float32),
            pltpu.VMEM_SHARED((n, d), jnp.float32),
            pltpu.SemaphoreType.DMA,
            pltpu.SemaphoreType.DMA,
            pltpu.SemaphoreType.DMA,
            pltpu.SemaphoreType.DMA,
            pltpu.SemaphoreType.DMA,
            pltpu.SemaphoreType.DMA,
            pltpu.SemaphoreType.DMA,
            pltpu.SemaphoreType.DMA,
        ],
    )
    def ek(k_hbm, q_hbm, v_hbm, src_hbm, dst_hbm, out_hbm,
           dst_v, src_v, kb, qb, vb, zb, agg,
           gk0, gk1, gq0, gq1, gv0, gv1, sc0, sc1):
        cid = lax.axis_index("c")
        sid = lax.axis_index("s")
        w = cid * ns + sid
        gk = (gk0, gk1)
        gq = (gq0, gq1)
        gv = (gv0, gv1)
        scs = (sc0, sc1)

        zero16 = jnp.zeros((16,), jnp.float32)

        def zero_row(r, carry):
            for g in range(d // 16):
                zb[r, pl.ds(g * 16, 16)] = zero16
            return carry
        lax.fori_loop(0, ZR, zero_row, 0)

        for j in range(nz):
            pltpu.sync_copy(zb, agg.at[pl.ds(sid * rpt + j * ZR, ZR)])
        if tail:
            @pl.when(sid == ns - 1)
            def _():
                pltpu.sync_copy(zb.at[pl.ds(0, tail)],
                                agg.at[pl.ds(rpt * ns, tail)])
        plsc.subcore_barrier()

        ebase = w * ept

        def load_idx(c, b):
            pltpu.sync_copy(dst_hbm.at[pl.ds(ebase + c * C, C)], dst_v.at[b])
            pltpu.sync_copy(src_hbm.at[pl.ds(ebase + c * C, C)], src_v.at[b])

        def start_gathers(b):
            ck = pltpu.async_copy(k_hbm.at[dst_v.at[b]], kb.at[b], gk[b])
            cq = pltpu.async_copy(q_hbm.at[src_v.at[b]], qb.at[b], gq[b])
            cv = pltpu.async_copy(v_hbm.at[src_v.at[b]], vb.at[b], gv[b])
            return ck, cq, cv

        def wait_gathers(b):
            pltpu.make_async_copy(k_hbm.at[dst_v.at[b]], kb.at[b], gk[b]).wait()
            pltpu.make_async_copy(q_hbm.at[src_v.at[b]], qb.at[b], gq[b]).wait()
            pltpu.make_async_copy(v_hbm.at[src_v.at[b]], vb.at[b], gv[b]).wait()

        def compute(b):
            def row(r, rcarry):
                for g in range(d // 16):
                    sl = pl.ds(g * 16, 16)
                    z = kb[b, r, sl] + qb[b, r, sl]
                    vb[b, r, sl] = vb[b, r, sl] / (1.0 + jnp.exp(-z))
                return rcarry
            lax.fori_loop(0, C, row, 0)

        def start_scatter(b):
            return pltpu.async_copy(vb.at[b], agg.at[dst_v.at[b]], scs[b],
                                    add=True)

        def wait_scatter(b):
            pltpu.make_async_copy(vb.at[b], agg.at[dst_v.at[b]], scs[b]).wait()

        load_idx(0, 0)
        start_gathers(0)

        def pair(j, carry):
            c0 = j * 2
            for b in range(2):
                c = c0 + b
                nb = 1 - b
                @pl.when(c >= 1)
                def _():
                    wait_scatter(nb)
                nxt = jnp.minimum(c + 1, nchunk - 1)
                load_idx(nxt, nb)
                start_gathers(nb)

                wait_gathers(b)
                compute(b)
                start_scatter(b)
            return carry
        lax.fori_loop(0, npair, pair, 0)

        wait_gathers(0)
        wait_scatter(1)

        plsc.subcore_barrier()

        for j in range(nz):
            off = sid * rpt + j * ZR
            pltpu.sync_copy(agg.at[pl.ds(off, ZR)], zb)
            pltpu.sync_copy(zb, out_hbm.at[cid, pl.ds(off, ZR)])
        if tail:
            @pl.when(sid == ns - 1)
            def _():
                pltpu.sync_copy(agg.at[pl.ds(rpt * ns, tail)],
                                zb.at[pl.ds(0, tail)])
                pltpu.sync_copy(zb.at[pl.ds(0, tail)],
                                out_hbm.at[cid, pl.ds(rpt * ns, tail)])

    return ek(k, q, v, src, dst)


def kernel(x, edge_index, Wk, bk, Wq, bq, Wv, bv, Ws, bs):
    n, d = x.shape
    k, q, v = _mm3(x, Wk, bk, Wq, bq, Wv, bv, 400)
    partials = _edge_sc(k, q, v, edge_index[0], edge_index[1])
    return _combine(x, Ws, bs, partials, 400)


# speedup vs baseline: 10.7040x; 1.6220x over previous
import functools

import jax
import jax.numpy as jnp
from jax import lax
from jax.experimental import pallas as pl
from jax.experimental.pallas import tpu as pltpu
from jax.experimental.pallas import tpu_sc as plsc


def _mm3_body(x_ref, wk_ref, bk_ref, wq_ref, bq_ref, wv_ref, bv_ref,
              k_ref, q_ref, v_ref):
    xb = x_ref[...]
    k_ref[...] = jnp.dot(xb, wk_ref[...], preferred_element_type=jnp.float32) + bk_ref[...]
    q_ref[...] = jnp.dot(xb, wq_ref[...], preferred_element_type=jnp.float32) + bq_ref[...]
    v_ref[...] = jnp.dot(xb, wv_ref[...], preferred_element_type=jnp.float32) + bv_ref[...]


def _mm3(x, Wk, bk, Wq, bq, Wv, bv, rb):
    n, d = x.shape
    grid = (n // rb,)
    row_bs = pl.BlockSpec((rb, d), lambda i: (i, 0))
    w_bs = pl.BlockSpec((d, d), lambda i: (0, 0))
    b_bs = pl.BlockSpec((1, d), lambda i: (0, 0))
    out_sd = jax.ShapeDtypeStruct((n, d), jnp.float32)
    return pl.pallas_call(
        _mm3_body,
        grid=grid,
        in_specs=[row_bs, w_bs, b_bs, w_bs, b_bs, w_bs, b_bs],
        out_specs=[row_bs, row_bs, row_bs],
        out_shape=[out_sd, out_sd, out_sd],
    )(x, Wk, bk.reshape(1, d), Wq, bq.reshape(1, d), Wv, bv.reshape(1, d))


def _combine_body(x_ref, ws_ref, bs_ref, p_ref, o_ref):
    o_ref[...] = (jnp.dot(x_ref[...], ws_ref[...], preferred_element_type=jnp.float32)
                  + bs_ref[...] + p_ref[0] + p_ref[1])


def _combine(x, Ws, bs, partials, rb):
    n, d = x.shape
    grid = (n // rb,)
    return pl.pallas_call(
        _combine_body,
        grid=grid,
        in_specs=[
            pl.BlockSpec((rb, d), lambda i: (i, 0)),
            pl.BlockSpec((d, d), lambda i: (0, 0)),
            pl.BlockSpec((1, d), lambda i: (0, 0)),
            pl.BlockSpec((2, rb, d), lambda i: (0, i, 0)),
        ],
        out_specs=pl.BlockSpec((rb, d), lambda i: (i, 0)),
        out_shape=jax.ShapeDtypeStruct((n, d), jnp.float32),
    )(x, Ws, bs.reshape(1, d), partials)


def _mm_pack_body(x_ref, wk_ref, bk_ref, wq_ref, bq_ref, wv_ref, bv_ref,
                  k2_ref, qv_ref):
    xb = x_ref[...]
    k2_ref[...] = jnp.dot(xb, wk_ref[...], preferred_element_type=jnp.float32) + bk_ref[...]
    q2 = jnp.dot(xb, wq_ref[...], preferred_element_type=jnp.float32) + bq_ref[...]
    v = jnp.dot(xb, wv_ref[...], preferred_element_type=jnp.float32) + bv_ref[...]
    qbits = jax.lax.bitcast_convert_type(
        q2.astype(jnp.bfloat16), jnp.uint16).astype(jnp.uint32)
    vbits = jax.lax.bitcast_convert_type(
        v.astype(jnp.bfloat16), jnp.uint16).astype(jnp.uint32)
    qv_ref[...] = jax.lax.bitcast_convert_type(
        (vbits << 16) | qbits, jnp.int32)


def _mm_pack(x, Wk2, bk2, Wq2, bq2, Wv, bv, rb):
    n, d = x.shape
    grid = (n // rb,)
    row_bs = pl.BlockSpec((rb, d), lambda i: (i, 0))
    w_bs = pl.BlockSpec((d, d), lambda i: (0, 0))
    b_bs = pl.BlockSpec((1, d), lambda i: (0, 0))
    return pl.pallas_call(
        _mm_pack_body,
        grid=grid,
        in_specs=[row_bs, w_bs, b_bs, w_bs, b_bs, w_bs, b_bs],
        out_specs=[row_bs, row_bs],
        out_shape=[jax.ShapeDtypeStruct((n, d), jnp.float32),
                   jax.ShapeDtypeStruct((n, d), jnp.int32)],
    )(x, Wk2, bk2.reshape(1, d), Wq2, bq2.reshape(1, d), Wv, bv.reshape(1, d))


def _edge_sc(k2, qv, src, dst):
    n, d = k2.shape
    e = src.shape[0]
    nc, ns = 2, 16
    nw = nc * ns
    ept = e // nw
    C = 40
    nchunk = ept // C
    npair = nchunk // 2
    UR = 4
    ZR = 48
    nz = 13
    rpt = ZR * nz
    tail = n - rpt * ns
    assert ept * nw == e and nchunk * C == ept and npair * 2 == nchunk
    assert C % UR == 0 and C % 8 == 0
    assert 0 <= tail <= ZR and tail % 8 == 0 and d % 16 == 0

    mesh = plsc.VectorSubcoreMesh(core_axis_name="c", subcore_axis_name="s")

    @functools.partial(
        pl.kernel, mesh=mesh,
        out_type=jax.ShapeDtypeStruct((nc, n, d), jnp.float32),
        scratch_types=[
            pltpu.VMEM((4, C), jnp.int32),
            pltpu.VMEM((4, C), jnp.int32),
            pltpu.VMEM((2, C, d), jnp.float32),
            pltpu.VMEM((2, C, d), jnp.int32),
            pltpu.VMEM((2, C, d), jnp.float32),
            pltpu.VMEM((ZR, d), jnp.float32),
            pltpu.VMEM_SHARED((n, d), jnp.float32),
            pltpu.SemaphoreType.DMA,
            pltpu.SemaphoreType.DMA,
            pltpu.SemaphoreType.DMA,
            pltpu.SemaphoreType.DMA,
            pltpu.SemaphoreType.DMA,
            pltpu.SemaphoreType.DMA,
            pltpu.SemaphoreType.DMA,
            pltpu.SemaphoreType.DMA,
            pltpu.SemaphoreType.DMA,
            pltpu.SemaphoreType.DMA,
            pltpu.SemaphoreType.DMA,
            pltpu.SemaphoreType.DMA,
            pltpu.SemaphoreType.DMA,
            pltpu.SemaphoreType.DMA,
        ],
    )
    def ek(k_hbm, qv_hbm, src_hbm, dst_hbm, out_hbm,
           dst_sv, src_sv, kb, qvb, mb, zb, agg,
           id0, id1, id2, id3, is0, is1, is2, is3,
           gk0, gk1, gq0, gq1, sc0, sc1):
        cid = lax.axis_index("c")
        sid = lax.axis_index("s")
        w = cid * ns + sid
        idx_d = (id0, id1, id2, id3)
        idx_s = (is0, is1, is2, is3)
        gk = (gk0, gk1)
        gq = (gq0, gq1)
        scs = (sc0, sc1)

        zero16 = jnp.zeros((16,), jnp.float32)

        def zero_row(r, carry):
            for g in range(d // 16):
                zb[r, pl.ds(g * 16, 16)] = zero16
            return carry
        lax.fori_loop(0, ZR, zero_row, 0)

        for j in range(nz):
            pltpu.sync_copy(zb, agg.at[pl.ds(sid * rpt + j * ZR, ZR)])
        if tail:
            @pl.when(sid == ns - 1)
            def _():
                pltpu.sync_copy(zb.at[pl.ds(0, tail)],
                                agg.at[pl.ds(rpt * ns, tail)])
        plsc.subcore_barrier()

        ebase = w * ept

        def start_idx(c, slot):
            base = ebase + c * C
            pltpu.async_copy(dst_hbm.at[pl.ds(base, C)], dst_sv.at[slot],
                             idx_d[slot])
            pltpu.async_copy(src_hbm.at[pl.ds(base, C)], src_sv.at[slot],
                             idx_s[slot])

        def wait_idx(slot):
            pltpu.make_async_copy(dst_hbm.at[pl.ds(0, C)], dst_sv.at[slot],
                                  idx_d[slot]).wait()
            pltpu.make_async_copy(src_hbm.at[pl.ds(0, C)], src_sv.at[slot],
                                  idx_s[slot]).wait()

        def start_gathers(b, slot):
            pltpu.async_copy(k_hbm.at[dst_sv.at[slot]], kb.at[b], gk[b])
            pltpu.async_copy(qv_hbm.at[src_sv.at[slot]], qvb.at[b], gq[b])

        def wait_gathers(b):
            pltpu.make_async_copy(k_hbm.at[dst_sv.at[0]], kb.at[b], gk[b]).wait()
            pltpu.make_async_copy(qv_hbm.at[src_sv.at[0]], qvb.at[b], gq[b]).wait()

        def compute(b):
            def rows(r4, rcarry):
                for u in range(UR):
                    r = r4 * UR + u
                    for g in range(d // 16):
                        sl = pl.ds(g * 16, 16)
                        wv = qvb[b, r, sl]
                        vg = jax.lax.bitcast_convert_type(
                            wv & jnp.int32(-65536), jnp.float32)
                        q2g = jax.lax.bitcast_convert_type(
                            wv << 16, jnp.float32)
                        z = kb[b, r, sl] + q2g
                        mb[b, r, sl] = vg / (1.0 + jnp.exp(z))
                return rcarry
            lax.fori_loop(0, C // UR, rows, 0)

        def start_scatter(b, slot):
            pltpu.async_copy(mb.at[b], agg.at[dst_sv.at[slot]], scs[b],
                             add=True)

        def wait_scatter(b):
            pltpu.make_async_copy(mb.at[b], agg.at[dst_sv.at[0]],
                                  scs[b]).wait()

        def start_idx_dyn(c, slot):
            for s_ in range(4):
                @pl.when(slot == s_)
                def _():
                    start_idx(c, s_)

        def wait_idx_dyn(slot):
            for s_ in range(4):
                @pl.when(slot == s_)
                def _():
                    wait_idx(s_)

        def start_gathers_dyn(b, slot):
            for s_ in range(4):
                @pl.when(slot == s_)
                def _():
                    start_gathers(b, s_)

        def start_scatter_dyn(b, slot):
            for s_ in range(4):
                @pl.when(slot == s_)
                def _():
                    start_scatter(b, s_)

        start_idx(0, 0)
        start_idx(1, 1)
        wait_idx(0)
        start_gathers(0, 0)

        last = nchunk - 1

        def pair(j, carry):
            m0 = j * 2
            for b in range(2):
                m = m0 + b
                nb = 1 - b
                @pl.when(m >= 2)
                def _():
                    wait_scatter(b)
                nn = jnp.minimum(m + 2, last)
                start_idx_dyn(nn, lax.rem(m + 2, 4))
                slot_g = lax.rem(m + 1, 4)
                wait_idx_dyn(slot_g)
                start_gathers_dyn(nb, slot_g)
                wait_gathers(b)
                compute(b)
                start_scatter_dyn(b, lax.rem(m, 4))
            return carry
        lax.fori_loop(0, npair, pair, 0)

        wait_gathers(0)
        wait_scatter(0)
        wait_scatter(1)
        wait_idx(3)

        plsc.subcore_barrier()

        for j in range(nz):
            off = sid * rpt + j * ZR
            pltpu.sync_copy(agg.at[pl.ds(off, ZR)], zb)
            pltpu.sync_copy(zb, out_hbm.at[cid, pl.ds(off, ZR)])
        if tail:
            @pl.when(sid == ns - 1)
            def _():
                pltpu.sync_copy(agg.at[pl.ds(rpt * ns, tail)],
                                zb.at[pl.ds(0, tail)])
                pltpu.sync_copy(zb.at[pl.ds(0, tail)],
                                out_hbm.at[cid, pl.ds(rpt * ns, tail)])

    return ek(k2, qv, src, dst)


def kernel(x, edge_index, Wk, bk, Wq, bq, Wv, bv, Ws, bs):
    n, d = x.shape
    k2, qv = _mm_pack(x, -Wk, -bk, -Wq, -bq, Wv, bv, 400)
    partials = _edge_sc(k2, qv, edge_index[0], edge_index[1])
    return _combine(x, Ws, bs, partials, 400)
